# baseline (speedup 1.0000x reference)
import jax
import jax.numpy as jnp
from jax.experimental import pallas as pl

EMBED = 2048
EXPERTS = 16
T_TILE = 256


def _gating_tile(x_ref, w_ref, b_ref, o_ref):
    w = w_ref[...]
    bias = b_ref[...]
    for bi in range(x_ref.shape[1]):
        xb = x_ref[:, bi, :]
        scores = jax.lax.dot_general(
            xb, w, (((1,), (1,)), ((), ())), preferred_element_type=jnp.float32
        )
        scores = scores + bias
        m = jnp.max(scores, axis=1, keepdims=True)
        e = jnp.exp(scores - m)
        o_ref[:, bi, :] = e / jnp.sum(e, axis=1, keepdims=True)


def kernel(x, W, b):
    target_length, batch_size, embed_dim = x.shape
    b2 = b.reshape(1, EXPERTS)
    grid = target_length // T_TILE
    return pl.pallas_call(
        _gating_tile,
        grid=(grid,),
        in_specs=[
            pl.BlockSpec((T_TILE, batch_size, embed_dim), lambda i: (i, 0, 0)),
            pl.BlockSpec((EXPERTS, embed_dim), lambda i: (0, 0)),
            pl.BlockSpec((1, EXPERTS), lambda i: (0, 0)),
        ],
        out_specs=pl.BlockSpec((T_TILE, batch_size, EXPERTS), lambda i: (i, 0, 0)),
        out_shape=jax.ShapeDtypeStruct(
            (target_length, batch_size, EXPERTS), jnp.float32
        ),
    )(x, W, b2)
